# Initial kernel scaffold; baseline (speedup 1.0000x reference)
import jax
import jax.numpy as jnp
from jax.experimental import pallas as pl


def kernel(x, edge_index):
    raise NotImplementedError("write your pallas kernel here")


# speedup vs baseline: 3.0610x; 3.0610x over previous
import functools

import jax
import jax.numpy as jnp
from jax import lax
from jax.experimental import pallas as pl
from jax.experimental.pallas import tpu as pltpu
from jax.experimental.pallas import tpu_sc as plsc

N_NODES = 10000
N_EDGES = 320000
D = 128
L = 16
NC = 2
NS = 16
NW = NC * NS
E_PER_W = N_EDGES // NW
C = 80
N_CHUNKS = E_PER_W // C
NV = D // L


def _sqrt16(s):
    i = lax.bitcast_convert_type(s, jnp.int32)
    i = jnp.int32(0x1FBD1DF5) + lax.shift_right_arithmetic(i, 1)
    y = lax.bitcast_convert_type(i, jnp.float32)
    half = jnp.float32(0.5)
    y = half * (y + s / y)
    y = half * (y + s / y)
    y = half * (y + s / y)
    return y


def _sc_body(x_hbm, src_hbm, dst_hbm, dist_hbm, dir_hbm,
             src_idx, dst_idx, src_rows, dst_rows, part, dist_v, recip_v,
             sem_a, sem_b):
    wid = lax.axis_index("s") * NC + lax.axis_index("c")
    base_w = wid * E_PER_W
    lane = lax.iota(jnp.int32, L)

    def chunk(j, carry):
        base = base_w + j * C
        pltpu.sync_copy(src_hbm.at[pl.ds(base, C)], src_idx)
        pltpu.sync_copy(dst_hbm.at[pl.ds(base, C)], dst_idx)
        cp_a = pltpu.async_copy(x_hbm.at[src_idx], src_rows, sem_a)
        cp_b = pltpu.async_copy(x_hbm.at[dst_idx], dst_rows, sem_b)
        cp_a.wait()
        cp_b.wait()

        def edge_diff(e, c):
            p = jnp.zeros((L,), jnp.float32)
            for v in range(NV):
                a = src_rows[e, pl.ds(v * L, L)]
                b = dst_rows[e, pl.ds(v * L, L)]
                d = a - b
                src_rows[e, pl.ds(v * L, L)] = d
                p = p + d * d
            part[e, :] = p
            return c

        lax.fori_loop(0, C, edge_diff, 0)

        def group(g, c):
            ids = g * L + lane
            s = jnp.zeros((L,), jnp.float32)
            for lj in range(L):
                col = jnp.full((L,), lj, jnp.int32)
                s = s + plsc.load_gather(part, [ids, col])
            dist = _sqrt16(s)
            dist_v[pl.ds(g * L, L)] = dist
            recip_v[pl.ds(g * L, L)] = jnp.float32(1.0) / (jnp.float32(1.0) + dist)
            return c

        lax.fori_loop(0, C // L, group, 0)

        def edge_scale(e, c):
            rv = plsc.load_gather(recip_v, [jnp.full((L,), e, jnp.int32)])
            for v in range(NV):
                src_rows[e, pl.ds(v * L, L)] = src_rows[e, pl.ds(v * L, L)] * rv
            return c

        lax.fori_loop(0, C, edge_scale, 0)

        pltpu.sync_copy(src_rows, dir_hbm.at[pl.ds(base, C)])
        pltpu.sync_copy(dist_v, dist_hbm.at[pl.ds(base, C)])
        return carry

    lax.fori_loop(0, N_CHUNKS, chunk, 0)


@jax.jit
def _add_edges_sc(x, src, dst):
    mesh = plsc.VectorSubcoreMesh(core_axis_name="c", subcore_axis_name="s")
    fn = pl.kernel(
        _sc_body,
        mesh=mesh,
        compiler_params=pltpu.CompilerParams(needs_layout_passes=False),
        out_type=[
            jax.ShapeDtypeStruct((N_EDGES,), jnp.float32),
            jax.ShapeDtypeStruct((N_EDGES, D), jnp.float32),
        ],
        scratch_types=[
            pltpu.VMEM((C,), jnp.int32),
            pltpu.VMEM((C,), jnp.int32),
            pltpu.VMEM((C, D), jnp.float32),
            pltpu.VMEM((C, D), jnp.float32),
            pltpu.VMEM((C, L), jnp.float32),
            pltpu.VMEM((C,), jnp.float32),
            pltpu.VMEM((C,), jnp.float32),
            pltpu.SemaphoreType.DMA,
            pltpu.SemaphoreType.DMA,
        ],
    )
    return fn(x, src, dst)


def kernel(x, edge_index):
    src = edge_index[0].astype(jnp.int32)
    dst = edge_index[1].astype(jnp.int32)
    dist, direction = _add_edges_sc(x, src, dst)
    return dist, direction


# speedup vs baseline: 6.3603x; 2.0779x over previous
import jax
import jax.numpy as jnp
from jax import lax
from jax.experimental import pallas as pl
from jax.experimental.pallas import tpu as pltpu
from jax.experimental.pallas import tpu_sc as plsc

N_NODES = 10000
N_EDGES = 320000
D = 128
L = 16
NC = 2
NS = 16
NW = NC * NS
E_PER_W = N_EDGES // NW
C = 80
N_CHUNKS = E_PER_W // C
NV = D // L


def _sqrt16(s):
    i = lax.bitcast_convert_type(s, jnp.int32)
    i = jnp.int32(0x1FBD1DF5) + lax.shift_right_arithmetic(i, 1)
    y = lax.bitcast_convert_type(i, jnp.float32)
    half = jnp.float32(0.5)
    y = half * (y + s / y)
    y = half * (y + s / y)
    y = half * (y + s / y)
    return y


def _sc_body(x_hbm, src_hbm, dst_hbm, dist_hbm, dir_hbm,
             src_idx0, src_idx1, dst_idx0, dst_idx1,
             src_rows0, src_rows1, dst_rows0, dst_rows1,
             dir_v0, dir_v1, part, dist_v0, dist_v1, recip_v,
             is0, is1, id0, id1, gs0, gs1, gd0, gd1, oa0, oa1, ob0, ob1):
    src_idx = (src_idx0, src_idx1)
    dst_idx = (dst_idx0, dst_idx1)
    src_rows = (src_rows0, src_rows1)
    dst_rows = (dst_rows0, dst_rows1)
    dir_v = (dir_v0, dir_v1)
    dist_v = (dist_v0, dist_v1)
    sem_is = (is0, is1)
    sem_id = (id0, id1)
    sem_gs = (gs0, gs1)
    sem_gd = (gd0, gd1)
    sem_oa = (oa0, oa1)
    sem_ob = (ob0, ob1)

    wid = lax.axis_index("s") * NC + lax.axis_index("c")
    base_w = wid * E_PER_W
    lane = lax.iota(jnp.int32, L)

    def ebase(c):
        return base_w + c * C

    def start_idx(c, b):
        pltpu.make_async_copy(
            src_hbm.at[pl.ds(ebase(c), C)], src_idx[b], sem_is[b]).start()
        pltpu.make_async_copy(
            dst_hbm.at[pl.ds(ebase(c), C)], dst_idx[b], sem_id[b]).start()

    def wait_idx(b):
        pltpu.make_async_copy(
            src_hbm.at[pl.ds(0, C)], src_idx[b], sem_is[b]).wait()
        pltpu.make_async_copy(
            dst_hbm.at[pl.ds(0, C)], dst_idx[b], sem_id[b]).wait()

    def start_gather(b):
        pltpu.make_async_copy(
            x_hbm.at[src_idx[b]], src_rows[b], sem_gs[b]).start()
        pltpu.make_async_copy(
            x_hbm.at[dst_idx[b]], dst_rows[b], sem_gd[b]).start()

    def wait_gather(b):
        pltpu.make_async_copy(
            x_hbm.at[src_idx[b]], src_rows[b], sem_gs[b]).wait()
        pltpu.make_async_copy(
            x_hbm.at[dst_idx[b]], dst_rows[b], sem_gd[b]).wait()

    def start_out(c, b):
        pltpu.make_async_copy(
            dir_v[b], dir_hbm.at[pl.ds(ebase(c), C)], sem_oa[b]).start()
        pltpu.make_async_copy(
            dist_v[b], dist_hbm.at[pl.ds(ebase(c), C)], sem_ob[b]).start()

    def wait_out(b):
        pltpu.make_async_copy(
            dir_v[b], dir_hbm.at[pl.ds(0, C)], sem_oa[b]).wait()
        pltpu.make_async_copy(
            dist_v[b], dist_hbm.at[pl.ds(0, C)], sem_ob[b]).wait()

    def compute(b):
        sr, dr, dv = src_rows[b], dst_rows[b], dir_v[b]

        def edge_diff(k, carry):
            for u in range(2):
                e = 2 * k + u
                p = jnp.zeros((L,), jnp.float32)
                for v in range(NV):
                    d = sr[e, pl.ds(v * L, L)] - dr[e, pl.ds(v * L, L)]
                    dv[e, pl.ds(v * L, L)] = d
                    p = p + d * d
                part[e, :] = p
            return carry

        lax.fori_loop(0, C // 2, edge_diff, 0)

        for g in range(C // L):
            ids = g * L + lane
            s = jnp.zeros((L,), jnp.float32)
            for lj in range(L):
                col = jnp.full((L,), lj, jnp.int32)
                s = s + plsc.load_gather(part, [ids, col])
            dist = _sqrt16(s)
            dist_v[b][pl.ds(g * L, L)] = dist
            recip_v[pl.ds(g * L, L)] = jnp.float32(1.0) / (jnp.float32(1.0) + dist)

        def edge_scale(k, carry):
            for u in range(2):
                e = 2 * k + u
                rv = plsc.load_gather(recip_v, [jnp.full((L,), e, jnp.int32)])
                for v in range(NV):
                    dv[e, pl.ds(v * L, L)] = dv[e, pl.ds(v * L, L)] * rv
            return carry

        lax.fori_loop(0, C // 2, edge_scale, 0)

    start_idx(0, 0)
    wait_idx(0)
    start_gather(0)
    start_idx(1, 1)

    def pair(j, carry):
        for b in range(2):
            c = 2 * j + b
            nb = 1 - b
            wait_idx(nb)
            start_gather(nb)
            wait_gather(b)
            @pl.when(c < N_CHUNKS - 2)
            def _():
                start_idx(c + 2, b)
            @pl.when(c >= 2)
            def _():
                wait_out(b)
            compute(b)
            start_out(c, b)
        return carry

    lax.fori_loop(0, (N_CHUNKS - 1) // 2, pair, 0)

    wait_gather(0)
    wait_out(0)
    compute(0)
    start_out(N_CHUNKS - 1, 0)
    wait_out(1)
    wait_out(0)


@jax.jit
def _add_edges_sc(x, src, dst):
    mesh = plsc.VectorSubcoreMesh(core_axis_name="c", subcore_axis_name="s")
    fn = pl.kernel(
        _sc_body,
        mesh=mesh,
        compiler_params=pltpu.CompilerParams(needs_layout_passes=False),
        out_type=[
            jax.ShapeDtypeStruct((N_EDGES,), jnp.float32),
            jax.ShapeDtypeStruct((N_EDGES, D), jnp.float32),
        ],
        scratch_types=[
            pltpu.VMEM((C,), jnp.int32),
            pltpu.VMEM((C,), jnp.int32),
            pltpu.VMEM((C,), jnp.int32),
            pltpu.VMEM((C,), jnp.int32),
            pltpu.VMEM((C, D), jnp.float32),
            pltpu.VMEM((C, D), jnp.float32),
            pltpu.VMEM((C, D), jnp.float32),
            pltpu.VMEM((C, D), jnp.float32),
            pltpu.VMEM((C, D), jnp.float32),
            pltpu.VMEM((C, D), jnp.float32),
            pltpu.VMEM((C, L), jnp.float32),
            pltpu.VMEM((C,), jnp.float32),
            pltpu.VMEM((C,), jnp.float32),
            pltpu.VMEM((C,), jnp.float32),
            pltpu.SemaphoreType.DMA,
            pltpu.SemaphoreType.DMA,
            pltpu.SemaphoreType.DMA,
            pltpu.SemaphoreType.DMA,
            pltpu.SemaphoreType.DMA,
            pltpu.SemaphoreType.DMA,
            pltpu.SemaphoreType.DMA,
            pltpu.SemaphoreType.DMA,
            pltpu.SemaphoreType.DMA,
            pltpu.SemaphoreType.DMA,
            pltpu.SemaphoreType.DMA,
            pltpu.SemaphoreType.DMA,
        ],
    )
    return fn(x, src, dst)


def kernel(x, edge_index):
    src = edge_index[0].astype(jnp.int32)
    dst = edge_index[1].astype(jnp.int32)
    dist, direction = _add_edges_sc(x, src, dst)
    return dist, direction


# speedup vs baseline: 7.1246x; 1.1202x over previous
import jax
import jax.numpy as jnp
from jax import lax
from jax.experimental import pallas as pl
from jax.experimental.pallas import tpu as pltpu
from jax.experimental.pallas import tpu_sc as plsc

N_NODES = 10000
N_EDGES = 320000
D = 128
L = 16
NC = 2
NS = 16
NW = NC * NS
E_PER_W = N_EDGES // NW
C = 80
N_CHUNKS = E_PER_W // C
NV = D // L


def _sqrt16(s):
    i = lax.bitcast_convert_type(s, jnp.int32)
    i = jnp.int32(0x1FBD1DF5) + lax.shift_right_arithmetic(i, 1)
    y = lax.bitcast_convert_type(i, jnp.float32)
    half = jnp.float32(0.5)
    y = half * (y + s / y)
    y = half * (y + s / y)
    y = half * (y + s / y)
    return y


def _sc_body(x_hbm, src_hbm, dst_hbm, dist_hbm, dir_hbm,
             src_idx0, src_idx1, dst_idx0, dst_idx1,
             src_rows0, src_rows1, dst_rows0, dst_rows1,
             dir_v0, dir_v1, part, dist_v0, dist_v1, recip_v,
             is0, is1, id0, id1, gs0, gs1, gd0, gd1, oa0, oa1, ob0, ob1):
    src_idx = (src_idx0, src_idx1)
    dst_idx = (dst_idx0, dst_idx1)
    src_rows = (src_rows0, src_rows1)
    dst_rows = (dst_rows0, dst_rows1)
    dir_v = (dir_v0, dir_v1)
    dist_v = (dist_v0, dist_v1)
    sem_is = (is0, is1)
    sem_id = (id0, id1)
    sem_gs = (gs0, gs1)
    sem_gd = (gd0, gd1)
    sem_oa = (oa0, oa1)
    sem_ob = (ob0, ob1)

    wid = lax.axis_index("s") * NC + lax.axis_index("c")
    base_w = wid * E_PER_W
    lane = lax.iota(jnp.int32, L)

    def ebase(c):
        return base_w + c * C

    def start_idx(c, b):
        pltpu.make_async_copy(
            src_hbm.at[pl.ds(ebase(c), C)], src_idx[b], sem_is[b]).start()
        pltpu.make_async_copy(
            dst_hbm.at[pl.ds(ebase(c), C)], dst_idx[b], sem_id[b]).start()

    def wait_idx(b):
        pltpu.make_async_copy(
            src_hbm.at[pl.ds(0, C)], src_idx[b], sem_is[b]).wait()
        pltpu.make_async_copy(
            dst_hbm.at[pl.ds(0, C)], dst_idx[b], sem_id[b]).wait()

    def start_gather(b):
        pltpu.make_async_copy(
            x_hbm.at[src_idx[b]], src_rows[b], sem_gs[b]).start()
        pltpu.make_async_copy(
            x_hbm.at[dst_idx[b]], dst_rows[b], sem_gd[b]).start()

    def wait_gather(b):
        pltpu.make_async_copy(
            x_hbm.at[src_idx[b]], src_rows[b], sem_gs[b]).wait()
        pltpu.make_async_copy(
            x_hbm.at[dst_idx[b]], dst_rows[b], sem_gd[b]).wait()

    def start_out(c, b):
        pltpu.make_async_copy(
            dir_v[b], dir_hbm.at[pl.ds(ebase(c), C)], sem_oa[b]).start()
        pltpu.make_async_copy(
            dist_v[b], dist_hbm.at[pl.ds(ebase(c), C)], sem_ob[b]).start()

    def wait_out(b):
        pltpu.make_async_copy(
            dir_v[b], dir_hbm.at[pl.ds(0, C)], sem_oa[b]).wait()
        pltpu.make_async_copy(
            dist_v[b], dist_hbm.at[pl.ds(0, C)], sem_ob[b]).wait()

    def compute(b):
        sr, dr, dv = src_rows[b], dst_rows[b], dir_v[b]

        @plsc.parallel_loop(0, C, 1, unroll=4)
        def edge_diff(e):
            p = jnp.zeros((L,), jnp.float32)
            for v in range(NV):
                d = sr[e, pl.ds(v * L, L)] - dr[e, pl.ds(v * L, L)]
                dv[e, pl.ds(v * L, L)] = d
                p = p + d * d
            part[e, :] = p

        for g in range(C // L):
            ids = g * L + lane
            s = jnp.zeros((L,), jnp.float32)
            for lj in range(L):
                col = jnp.full((L,), lj, jnp.int32)
                s = s + plsc.load_gather(part, [ids, col])
            dist = _sqrt16(s)
            dist_v[b][pl.ds(g * L, L)] = dist
            recip_v[pl.ds(g * L, L)] = jnp.float32(1.0) / (jnp.float32(1.0) + dist)

        @plsc.parallel_loop(0, C, 1, unroll=4)
        def edge_scale(e):
            rv = plsc.load_gather(recip_v, [jnp.full((L,), e, jnp.int32)])
            for v in range(NV):
                dv[e, pl.ds(v * L, L)] = dv[e, pl.ds(v * L, L)] * rv

    start_idx(0, 0)
    wait_idx(0)
    start_gather(0)
    start_idx(1, 1)

    def pair(j, carry):
        for b in range(2):
            c = 2 * j + b
            nb = 1 - b
            wait_idx(nb)
            start_gather(nb)
            wait_gather(b)
            @pl.when(c < N_CHUNKS - 2)
            def _():
                start_idx(c + 2, b)
            @pl.when(c >= 2)
            def _():
                wait_out(b)
            compute(b)
            start_out(c, b)
        return carry

    lax.fori_loop(0, (N_CHUNKS - 1) // 2, pair, 0)

    wait_gather(0)
    wait_out(0)
    compute(0)
    start_out(N_CHUNKS - 1, 0)
    wait_out(1)
    wait_out(0)


@jax.jit
def _add_edges_sc(x, src, dst):
    mesh = plsc.VectorSubcoreMesh(core_axis_name="c", subcore_axis_name="s")
    fn = pl.kernel(
        _sc_body,
        mesh=mesh,
        compiler_params=pltpu.CompilerParams(needs_layout_passes=False),
        out_type=[
            jax.ShapeDtypeStruct((N_EDGES,), jnp.float32),
            jax.ShapeDtypeStruct((N_EDGES, D), jnp.float32),
        ],
        scratch_types=[
            pltpu.VMEM((C,), jnp.int32),
            pltpu.VMEM((C,), jnp.int32),
            pltpu.VMEM((C,), jnp.int32),
            pltpu.VMEM((C,), jnp.int32),
            pltpu.VMEM((C, D), jnp.float32),
            pltpu.VMEM((C, D), jnp.float32),
            pltpu.VMEM((C, D), jnp.float32),
            pltpu.VMEM((C, D), jnp.float32),
            pltpu.VMEM((C, D), jnp.float32),
            pltpu.VMEM((C, D), jnp.float32),
            pltpu.VMEM((C, L), jnp.float32),
            pltpu.VMEM((C,), jnp.float32),
            pltpu.VMEM((C,), jnp.float32),
            pltpu.VMEM((C,), jnp.float32),
            pltpu.SemaphoreType.DMA,
            pltpu.SemaphoreType.DMA,
            pltpu.SemaphoreType.DMA,
            pltpu.SemaphoreType.DMA,
            pltpu.SemaphoreType.DMA,
            pltpu.SemaphoreType.DMA,
            pltpu.SemaphoreType.DMA,
            pltpu.SemaphoreType.DMA,
            pltpu.SemaphoreType.DMA,
            pltpu.SemaphoreType.DMA,
            pltpu.SemaphoreType.DMA,
            pltpu.SemaphoreType.DMA,
        ],
    )
    return fn(x, src, dst)


def kernel(x, edge_index):
    src = edge_index[0].astype(jnp.int32)
    dst = edge_index[1].astype(jnp.int32)
    dist, direction = _add_edges_sc(x, src, dst)
    return dist, direction


# speedup vs baseline: 7.9367x; 1.1140x over previous
import jax
import jax.numpy as jnp
from jax import lax
from jax.experimental import pallas as pl
from jax.experimental.pallas import tpu as pltpu
from jax.experimental.pallas import tpu_sc as plsc

N_NODES = 10000
N_EDGES = 320000
D = 128
L = 16
NC = 2
NS = 16
NW = NC * NS
E_PER_W = N_EDGES // NW
C = 80
N_CHUNKS = E_PER_W // C
NV = D // L


def _sc_body(x_hbm, src_hbm, dst_hbm, dist_hbm, dir_hbm,
             src_idx0, src_idx1, dst_idx0, dst_idx1,
             src_rows0, src_rows1, dst_rows0, dst_rows1,
             dir_v0, dir_v1, dist_v0, dist_v1,
             is0, is1, id0, id1, gs0, gs1, gd0, gd1, oa0, oa1, ob0, ob1):
    src_idx = (src_idx0, src_idx1)
    dst_idx = (dst_idx0, dst_idx1)
    src_rows = (src_rows0, src_rows1)
    dst_rows = (dst_rows0, dst_rows1)
    dir_v = (dir_v0, dir_v1)
    dist_v = (dist_v0, dist_v1)
    sem_is = (is0, is1)
    sem_id = (id0, id1)
    sem_gs = (gs0, gs1)
    sem_gd = (gd0, gd1)
    sem_oa = (oa0, oa1)
    sem_ob = (ob0, ob1)

    wid = lax.axis_index("s") * NC + lax.axis_index("c")
    base_w = wid * E_PER_W
    lane = lax.iota(jnp.int32, L)

    def ebase(c):
        return base_w + c * C

    def start_idx(c, b):
        pltpu.make_async_copy(
            src_hbm.at[pl.ds(ebase(c), C)], src_idx[b], sem_is[b]).start()
        pltpu.make_async_copy(
            dst_hbm.at[pl.ds(ebase(c), C)], dst_idx[b], sem_id[b]).start()

    def wait_idx(b):
        pltpu.make_async_copy(
            src_hbm.at[pl.ds(0, C)], src_idx[b], sem_is[b]).wait()
        pltpu.make_async_copy(
            dst_hbm.at[pl.ds(0, C)], dst_idx[b], sem_id[b]).wait()

    def start_gather(b):
        pltpu.make_async_copy(
            x_hbm.at[src_idx[b]], src_rows[b], sem_gs[b]).start()
        pltpu.make_async_copy(
            x_hbm.at[dst_idx[b]], dst_rows[b], sem_gd[b]).start()

    def wait_gather(b):
        pltpu.make_async_copy(
            x_hbm.at[src_idx[b]], src_rows[b], sem_gs[b]).wait()
        pltpu.make_async_copy(
            x_hbm.at[dst_idx[b]], dst_rows[b], sem_gd[b]).wait()

    def start_out(c, b):
        pltpu.make_async_copy(
            dir_v[b], dir_hbm.at[pl.ds(ebase(c), C)], sem_oa[b]).start()
        pltpu.make_async_copy(
            dist_v[b], dist_hbm.at[pl.ds(ebase(c), C)], sem_ob[b]).start()

    def wait_out(b):
        pltpu.make_async_copy(
            dir_v[b], dir_hbm.at[pl.ds(0, C)], sem_oa[b]).wait()
        pltpu.make_async_copy(
            dist_v[b], dist_hbm.at[pl.ds(0, C)], sem_ob[b]).wait()

    G = 4
    LG = L // G
    shuf = lambda v, perm: jnp.take_along_axis(
        v, perm, axis=0, mode="promise_in_bounds")
    perms = [lane ^ jnp.int32(1 << t) for t in range(4)]
    bperms = [jnp.full((L,), G * u, jnp.int32) for u in range(G)]
    emask = [(lane >> 2) == u for u in range(G)]
    dmask = (lane & 3) == 0
    dlane = lane >> 2

    def compute(b):
        sr, dr, dv = src_rows[b], dst_rows[b], dir_v[b]

        @plsc.parallel_loop(0, C // G, 1, unroll=1)
        def edge_group(k):
            e0 = G * k
            diffs = []
            merged = jnp.zeros((L,), jnp.float32)
            for u in range(G):
                e = e0 + u
                du = []
                p = None
                for v in range(NV):
                    d = sr[e, pl.ds(v * L, L)] - dr[e, pl.ds(v * L, L)]
                    du.append(d)
                    p = d * d if p is None else p + d * d
                for t in range(4):
                    p = p + shuf(p, perms[t])
                merged = jnp.where(emask[u], p, merged)
                diffs.append(du)
            m = jnp.maximum(merged, jnp.float32(1e-30))
            i = lax.bitcast_convert_type(m, jnp.int32)
            i = jnp.int32(0x5F3759DF) - lax.shift_right_arithmetic(i, 1)
            y = lax.bitcast_convert_type(i, jnp.float32)
            hm = jnp.float32(0.5) * m
            for _ in range(2):
                t = y * y
                t = hm * t
                y = y * (jnp.float32(1.5) - t)
            dist = m * y
            a = jnp.float32(1.0) + dist
            i = lax.bitcast_convert_type(a, jnp.int32)
            i = jnp.int32(0x7EF311C3) - i
            z = lax.bitcast_convert_type(i, jnp.float32)
            for _ in range(3):
                z = z * (jnp.float32(2.0) - a * z)
            plsc.store_scatter(dist_v[b], [e0 + dlane], dist, mask=dmask)
            for u in range(G):
                rv = shuf(z, bperms[u])
                for v in range(NV):
                    dv[e0 + u, pl.ds(v * L, L)] = diffs[u][v] * rv

    start_idx(0, 0)
    wait_idx(0)
    start_gather(0)
    start_idx(1, 1)

    def pair(j, carry):
        for b in range(2):
            c = 2 * j + b
            nb = 1 - b
            wait_idx(nb)
            start_gather(nb)
            wait_gather(b)
            @pl.when(c < N_CHUNKS - 2)
            def _():
                start_idx(c + 2, b)
            @pl.when(c >= 2)
            def _():
                wait_out(b)
            compute(b)
            start_out(c, b)
        return carry

    lax.fori_loop(0, (N_CHUNKS - 1) // 2, pair, 0)

    wait_gather(0)
    wait_out(0)
    compute(0)
    start_out(N_CHUNKS - 1, 0)
    wait_out(1)
    wait_out(0)


@jax.jit
def _add_edges_sc(x, src, dst):
    mesh = plsc.VectorSubcoreMesh(core_axis_name="c", subcore_axis_name="s")
    fn = pl.kernel(
        _sc_body,
        mesh=mesh,
        compiler_params=pltpu.CompilerParams(needs_layout_passes=False),
        out_type=[
            jax.ShapeDtypeStruct((N_EDGES,), jnp.float32),
            jax.ShapeDtypeStruct((N_EDGES, D), jnp.float32),
        ],
        scratch_types=[
            pltpu.VMEM((C,), jnp.int32),
            pltpu.VMEM((C,), jnp.int32),
            pltpu.VMEM((C,), jnp.int32),
            pltpu.VMEM((C,), jnp.int32),
            pltpu.VMEM((C, D), jnp.float32),
            pltpu.VMEM((C, D), jnp.float32),
            pltpu.VMEM((C, D), jnp.float32),
            pltpu.VMEM((C, D), jnp.float32),
            pltpu.VMEM((C, D), jnp.float32),
            pltpu.VMEM((C, D), jnp.float32),
            pltpu.VMEM((C,), jnp.float32),
            pltpu.VMEM((C,), jnp.float32),
            pltpu.SemaphoreType.DMA,
            pltpu.SemaphoreType.DMA,
            pltpu.SemaphoreType.DMA,
            pltpu.SemaphoreType.DMA,
            pltpu.SemaphoreType.DMA,
            pltpu.SemaphoreType.DMA,
            pltpu.SemaphoreType.DMA,
            pltpu.SemaphoreType.DMA,
            pltpu.SemaphoreType.DMA,
            pltpu.SemaphoreType.DMA,
            pltpu.SemaphoreType.DMA,
            pltpu.SemaphoreType.DMA,
        ],
    )
    return fn(x, src, dst)


def kernel(x, edge_index):
    src = edge_index[0].astype(jnp.int32)
    dst = edge_index[1].astype(jnp.int32)
    dist, direction = _add_edges_sc(x, src, dst)
    return dist, direction


# speedup vs baseline: 8.9200x; 1.1239x over previous
import jax
import jax.numpy as jnp
from jax import lax
from jax.experimental import pallas as pl
from jax.experimental.pallas import tpu as pltpu
from jax.experimental.pallas import tpu_sc as plsc

N_NODES = 10000
N_EDGES = 320000
D = 128
L = 16
NC = 2
NS = 16
NW = NC * NS
E_PER_W = N_EDGES // NW
C = 80
N_CHUNKS = E_PER_W // C
NV = D // L


def _sc_body(x_hbm, src_hbm, dst_hbm, dist_hbm, dir_hbm,
             src_idx0, src_idx1, dst_idx0, dst_idx1,
             src_rows0, src_rows1, dst_rows0, dst_rows1,
             dir_v0, dir_v1, dist_v0, dist_v1,
             is0, is1, id0, id1, gs0, gs1, gd0, gd1, oa0, oa1, ob0, ob1):
    src_idx = (src_idx0, src_idx1)
    dst_idx = (dst_idx0, dst_idx1)
    src_rows = (src_rows0, src_rows1)
    dst_rows = (dst_rows0, dst_rows1)
    dir_v = (dir_v0, dir_v1)
    dist_v = (dist_v0, dist_v1)
    sem_is = (is0, is1)
    sem_id = (id0, id1)
    sem_gs = (gs0, gs1)
    sem_gd = (gd0, gd1)
    sem_oa = (oa0, oa1)
    sem_ob = (ob0, ob1)

    wid = lax.axis_index("s") * NC + lax.axis_index("c")
    base_w = wid * E_PER_W
    lane = lax.iota(jnp.int32, L)

    def ebase(c):
        return base_w + c * C

    def start_idx(c, b):
        pltpu.make_async_copy(
            src_hbm.at[pl.ds(ebase(c), C)], src_idx[b], sem_is[b]).start()
        pltpu.make_async_copy(
            dst_hbm.at[pl.ds(ebase(c), C)], dst_idx[b], sem_id[b]).start()

    def wait_idx(b):
        pltpu.make_async_copy(
            src_hbm.at[pl.ds(0, C)], src_idx[b], sem_is[b]).wait()
        pltpu.make_async_copy(
            dst_hbm.at[pl.ds(0, C)], dst_idx[b], sem_id[b]).wait()

    def start_gather(b):
        pltpu.make_async_copy(
            x_hbm.at[src_idx[b]], src_rows[b], sem_gs[b]).start()
        pltpu.make_async_copy(
            x_hbm.at[dst_idx[b]], dst_rows[b], sem_gd[b]).start()

    def wait_gather(b):
        pltpu.make_async_copy(
            x_hbm.at[src_idx[b]], src_rows[b], sem_gs[b]).wait()
        pltpu.make_async_copy(
            x_hbm.at[dst_idx[b]], dst_rows[b], sem_gd[b]).wait()

    def start_out(c, b):
        pltpu.make_async_copy(
            dir_v[b], dir_hbm.at[pl.ds(ebase(c), C)], sem_oa[b]).start()
        pltpu.make_async_copy(
            dist_v[b], dist_hbm.at[pl.ds(ebase(c), C)], sem_ob[b]).start()

    def wait_out(b):
        pltpu.make_async_copy(
            dir_v[b], dir_hbm.at[pl.ds(0, C)], sem_oa[b]).wait()
        pltpu.make_async_copy(
            dist_v[b], dist_hbm.at[pl.ds(0, C)], sem_ob[b]).wait()

    G = 2
    LG = L // G
    shuf = lambda v, perm: jnp.take_along_axis(
        v, perm, axis=0, mode="promise_in_bounds")
    perms = [lane ^ jnp.int32(1 << t) for t in range(4)]
    bperms = [jnp.full((L,), (L // G) * u, jnp.int32) for u in range(G)]
    emask = [(lane >> 3) == u for u in range(G)]
    dmask = (lane & 7) == 0
    dlane = lane >> 3

    def compute(b):
        sr, dr, dv = src_rows[b], dst_rows[b], dir_v[b]

        @plsc.parallel_loop(0, C // G, 1, unroll=1)
        def edge_group(k):
            e0 = G * k
            diffs = []
            merged = jnp.zeros((L,), jnp.float32)
            for u in range(G):
                e = e0 + u
                du = []
                p = None
                for v in range(NV):
                    d = sr[e, pl.ds(v * L, L)] - dr[e, pl.ds(v * L, L)]
                    du.append(d)
                    p = d * d if p is None else p + d * d
                for t in range(4):
                    p = p + shuf(p, perms[t])
                merged = jnp.where(emask[u], p, merged)
                diffs.append(du)
            m = jnp.maximum(merged, jnp.float32(1e-30))
            i = lax.bitcast_convert_type(m, jnp.int32)
            i = jnp.int32(0x5F3759DF) - lax.shift_right_arithmetic(i, 1)
            y = lax.bitcast_convert_type(i, jnp.float32)
            hm = jnp.float32(0.5) * m
            for _ in range(2):
                t = y * y
                t = hm * t
                y = y * (jnp.float32(1.5) - t)
            dist = m * y
            a = jnp.float32(1.0) + dist
            i = lax.bitcast_convert_type(a, jnp.int32)
            i = jnp.int32(0x7EF311C3) - i
            z = lax.bitcast_convert_type(i, jnp.float32)
            for _ in range(3):
                z = z * (jnp.float32(2.0) - a * z)
            plsc.store_scatter(dist_v[b], [e0 + dlane], dist, mask=dmask)
            for u in range(G):
                rv = shuf(z, bperms[u])
                for v in range(NV):
                    dv[e0 + u, pl.ds(v * L, L)] = diffs[u][v] * rv

    start_idx(0, 0)
    wait_idx(0)
    start_gather(0)
    start_idx(1, 1)

    def pair(j, carry):
        for b in range(2):
            c = 2 * j + b
            nb = 1 - b
            wait_idx(nb)
            start_gather(nb)
            wait_gather(b)
            @pl.when(c < N_CHUNKS - 2)
            def _():
                start_idx(c + 2, b)
            @pl.when(c >= 2)
            def _():
                wait_out(b)
            compute(b)
            start_out(c, b)
        return carry

    lax.fori_loop(0, (N_CHUNKS - 1) // 2, pair, 0)

    wait_gather(0)
    wait_out(0)
    compute(0)
    start_out(N_CHUNKS - 1, 0)
    wait_out(1)
    wait_out(0)


@jax.jit
def _add_edges_sc(x, src, dst):
    mesh = plsc.VectorSubcoreMesh(core_axis_name="c", subcore_axis_name="s")
    fn = pl.kernel(
        _sc_body,
        mesh=mesh,
        compiler_params=pltpu.CompilerParams(needs_layout_passes=False),
        out_type=[
            jax.ShapeDtypeStruct((N_EDGES,), jnp.float32),
            jax.ShapeDtypeStruct((N_EDGES, D), jnp.float32),
        ],
        scratch_types=[
            pltpu.VMEM((C,), jnp.int32),
            pltpu.VMEM((C,), jnp.int32),
            pltpu.VMEM((C,), jnp.int32),
            pltpu.VMEM((C,), jnp.int32),
            pltpu.VMEM((C, D), jnp.float32),
            pltpu.VMEM((C, D), jnp.float32),
            pltpu.VMEM((C, D), jnp.float32),
            pltpu.VMEM((C, D), jnp.float32),
            pltpu.VMEM((C, D), jnp.float32),
            pltpu.VMEM((C, D), jnp.float32),
            pltpu.VMEM((C,), jnp.float32),
            pltpu.VMEM((C,), jnp.float32),
            pltpu.SemaphoreType.DMA,
            pltpu.SemaphoreType.DMA,
            pltpu.SemaphoreType.DMA,
            pltpu.SemaphoreType.DMA,
            pltpu.SemaphoreType.DMA,
            pltpu.SemaphoreType.DMA,
            pltpu.SemaphoreType.DMA,
            pltpu.SemaphoreType.DMA,
            pltpu.SemaphoreType.DMA,
            pltpu.SemaphoreType.DMA,
            pltpu.SemaphoreType.DMA,
            pltpu.SemaphoreType.DMA,
        ],
    )
    return fn(x, src, dst)


def kernel(x, edge_index):
    src = edge_index[0].astype(jnp.int32)
    dst = edge_index[1].astype(jnp.int32)
    dist, direction = _add_edges_sc(x, src, dst)
    return dist, direction
